# baseline (speedup 1.0000x reference)
import functools

import jax
import jax.numpy as jnp
from jax import lax
from jax.experimental import pallas as pl
from jax.experimental.pallas import tpu as pltpu
from jax.experimental.pallas import tpu_sc as plsc

NSC = 2
NSUB = 16
NTILES = NSC * NSUB
CS = 128
G = 8
NOB = 10


def _pad_nodes(n):
    return ((n + 128) // 128) * 128


def _pad_nodes_deg(n):
    return ((n + 255) // 256) * 256


def _mesh():
    return plsc.VectorSubcoreMesh(core_axis_name="c", subcore_axis_name="s")


def _sc_degrees(src4, dst4, ones_h, npad):
    rows_pt = npad // NSUB

    @functools.partial(
        pl.kernel,
        out_type=(jax.ShapeDtypeStruct((npad,), jnp.float32),
                  jax.ShapeDtypeStruct((npad,), jnp.float32),
                  jax.ShapeDtypeStruct((npad,), jnp.float32),
                  jax.ShapeDtypeStruct((npad,), jnp.float32)),
        mesh=_mesh(),
        scratch_types=[
            pltpu.VMEM((G, CS), jnp.int32),
            pltpu.VMEM((G, CS), jnp.int32),
            pltpu.VMEM((G, CS), jnp.int32),
            pltpu.VMEM((G, CS), jnp.int32),
            pltpu.VMEM((CS,), jnp.float32),
            pltpu.VMEM((((npad // NSUB + 15) // 16) * 16,), jnp.float32),
            pltpu.VMEM_SHARED((npad,), jnp.float32),
            pltpu.VMEM_SHARED((npad,), jnp.float32),
            pltpu.SemaphoreType.DMA,
            pltpu.SemaphoreType.DMA,
            pltpu.SemaphoreType.DMA,
        ],
    )
    def deg_kernel(src_hbm, dst_hbm, ones_hbm,
                   dout0_hbm, dout1_hbm, din0_hbm, din1_hbm,
                   sidx0, didx0, sidx1, didx1, ones, zbuf, acc_o, acc_i,
                   isem0, isem1, ssem):
        c = lax.axis_index("c")
        s = lax.axis_index("s")
        t = s * NSC + c
        idxp = ((sidx0, didx0, isem0), (sidx1, didx1, isem1))
        pltpu.async_copy(src_hbm.at[t, 0], sidx0, isem0)
        pltpu.async_copy(dst_hbm.at[t, 0], didx0, isem0)
        pltpu.async_copy(src_hbm.at[t, 1], sidx1, isem1)
        pltpu.async_copy(dst_hbm.at[t, 1], didx1, isem1)
        pltpu.sync_copy(ones_hbm, ones)

        def init_z(i, _):
            zbuf[pl.ds(i * 16, 16)] = jnp.zeros((16,), jnp.float32)
            return 0
        lax.fori_loop(0, zbuf.shape[0] // 16, init_z, 0)
        pltpu.sync_copy(zbuf.at[pl.ds(0, rows_pt)],
                        acc_o.at[pl.ds(s * rows_pt, rows_pt)])
        pltpu.sync_copy(zbuf.at[pl.ds(0, rows_pt)],
                        acc_i.at[pl.ds(s * rows_pt, rows_pt)])
        plsc.subcore_barrier()

        def process_block(ob, p):
            sidx, didx, isem = idxp[p]
            pltpu.make_async_copy(src_hbm.at[t, 0], sidx, isem).wait()
            pltpu.make_async_copy(dst_hbm.at[t, 0], didx, isem).wait()
            for g in range(G):
                pltpu.async_copy(ones, acc_o.at[sidx.at[g]], ssem, add=True)
                pltpu.async_copy(ones, acc_i.at[didx.at[g]], ssem, add=True)
            for g in range(G):
                pltpu.make_async_copy(ones, acc_o.at[sidx.at[0]], ssem).wait()
                pltpu.make_async_copy(ones, acc_i.at[didx.at[0]], ssem).wait()
            @pl.when(ob + 2 < NOB)
            def _():
                pltpu.async_copy(src_hbm.at[t, ob + 2], sidx, isem)
                pltpu.async_copy(dst_hbm.at[t, ob + 2], didx, isem)

        def body(i, _):
            process_block(2 * i, 0)
            process_block(2 * i + 1, 1)
            return 0
        lax.fori_loop(0, NOB // 2, body, 0)
        plsc.subcore_barrier()

        @pl.when(c == 0)
        def _():
            pltpu.sync_copy(acc_o.at[pl.ds(s * rows_pt, rows_pt)],
                            dout0_hbm.at[pl.ds(s * rows_pt, rows_pt)])
            pltpu.sync_copy(acc_i.at[pl.ds(s * rows_pt, rows_pt)],
                            din0_hbm.at[pl.ds(s * rows_pt, rows_pt)])

        @pl.when(c == 1)
        def _():
            pltpu.sync_copy(acc_o.at[pl.ds(s * rows_pt, rows_pt)],
                            dout1_hbm.at[pl.ds(s * rows_pt, rows_pt)])
            pltpu.sync_copy(acc_i.at[pl.ds(s * rows_pt, rows_pt)],
                            din1_hbm.at[pl.ds(s * rows_pt, rows_pt)])

    return deg_kernel(src4, dst4, ones_h)


def _sc_spmm(z, src4, dst4, zeros2_h, npad):
    d = z.shape[1]
    rows_pt = npad // NSUB

    @functools.partial(
        pl.kernel,
        out_type=jax.ShapeDtypeStruct((NSC, npad, d), jnp.float32),
        mesh=_mesh(),
        scratch_types=[
            pltpu.VMEM((G, CS), jnp.int32),
            pltpu.VMEM((G, CS), jnp.int32),
            pltpu.VMEM((G, CS), jnp.int32),
            pltpu.VMEM((G, CS), jnp.int32),
            pltpu.VMEM((CS, d), jnp.float32),
            pltpu.VMEM((CS, d), jnp.float32),
            pltpu.VMEM_SHARED((npad, d), jnp.float32),
            pltpu.SemaphoreType.DMA,
            pltpu.SemaphoreType.DMA,
            pltpu.SemaphoreType.DMA,
            pltpu.SemaphoreType.DMA,
            pltpu.SemaphoreType.DMA,
            pltpu.SemaphoreType.DMA,
        ],
    )
    def spmm_kernel(z_hbm, src_hbm, dst_hbm, zeros_hbm, out_hbm,
                    sidx0, didx0, sidx1, didx1, rb0, rb1, acc,
                    sem0, sem1, isem0, isem1, ssem0, ssem1):
        c = lax.axis_index("c")
        s = lax.axis_index("s")
        t = s * NSC + c
        rbs = (rb0, rb1)
        sems = (sem0, sem1)
        ssems = (ssem0, ssem1)
        idxp = ((sidx0, didx0, isem0), (sidx1, didx1, isem1))

        def wait_scatter(b, didx):
            pltpu.make_async_copy(rbs[b], acc.at[didx.at[0]], ssems[b]).wait()

        pltpu.async_copy(src_hbm.at[t, 0], sidx0, isem0)
        pltpu.async_copy(dst_hbm.at[t, 0], didx0, isem0)
        pltpu.async_copy(src_hbm.at[t, 1], sidx1, isem1)
        pltpu.async_copy(dst_hbm.at[t, 1], didx1, isem1)

        def zc(i, _):
            pltpu.sync_copy(zeros_hbm,
                            acc.at[pl.ds(s * rows_pt + i * 128, 128)])
            return 0
        lax.fori_loop(0, rows_pt // 128, zc, 0)
        rem = rows_pt % 128
        if rem:
            pltpu.sync_copy(
                zeros_hbm.at[pl.ds(0, rem)],
                acc.at[pl.ds(s * rows_pt + (rows_pt - rem), rem)])
        plsc.subcore_barrier()

        def wait_idx(sidx, didx, isem):
            pltpu.make_async_copy(src_hbm.at[t, 0], sidx, isem).wait()
            pltpu.make_async_copy(dst_hbm.at[t, 0], didx, isem).wait()

        def process_block(ob, p):
            sidx, didx, isem = idxp[p]
            wait_idx(sidx, didx, isem)

            cps = [None, None]
            cps[0] = pltpu.async_copy(z_hbm.at[sidx.at[0]], rb0, sem0)
            for g in range(G):
                b = g % 2
                if g + 1 < G:
                    nb = (g + 1) % 2
                    if g >= 1:
                        wait_scatter(nb, didx)
                    cps[nb] = pltpu.async_copy(
                        z_hbm.at[sidx.at[g + 1]], rbs[nb], sems[nb])
                cps[b].wait()
                pltpu.async_copy(rbs[b], acc.at[didx.at[g]], ssems[b],
                                 add=True)
            wait_scatter(0, didx)
            wait_scatter(1, didx)
            @pl.when(ob + 2 < NOB)
            def _():
                pltpu.async_copy(src_hbm.at[t, ob + 2], sidx, isem)
                pltpu.async_copy(dst_hbm.at[t, ob + 2], didx, isem)

        def body(i, _):
            process_block(2 * i, 0)
            process_block(2 * i + 1, 1)
            return 0
        lax.fori_loop(0, NOB // 2, body, 0)
        plsc.subcore_barrier()
        pltpu.sync_copy(acc.at[pl.ds(s * rows_pt, rows_pt)],
                        out_hbm.at[c, pl.ds(s * rows_pt, rows_pt)])

    return spmm_kernel(z, src4, dst4, zeros2_h)


def _gelu(x):
    return 0.5 * x * (1.0 + lax.erf(x * (2.0 ** -0.5)))


def _tc1(do0, do1, di0, di1, x, w1, npad):
    n, d = x.shape

    def body(do0_ref, do1_ref, di0_ref, di1_ref, x_ref, w1_ref,
             z1_ref, ns_ref, nd_ref):
        od = do0_ref[...] + do1_ref[...]
        idg = di0_ref[...] + di1_ref[...]
        ns = jnp.where(od > 0, lax.rsqrt(jnp.maximum(od, 1.0)), 0.0)[:n]
        nd = jnp.where(idg > 0, lax.rsqrt(jnp.maximum(idg, 1.0)), 0.0)[:n]
        y = jnp.dot(x_ref[...], w1_ref[...], preferred_element_type=jnp.float32)
        z1_ref[...] = jnp.concatenate(
            [y * ns[:, None], jnp.zeros((npad - n, d), jnp.float32)], axis=0)
        ns_ref[...] = ns
        nd_ref[...] = nd

    return pl.pallas_call(
        body,
        out_shape=(jax.ShapeDtypeStruct((npad, d), jnp.float32),
                   jax.ShapeDtypeStruct((n,), jnp.float32),
                   jax.ShapeDtypeStruct((n,), jnp.float32)),
    )(do0, do1, di0, di1, x, w1)


def _tc2(p, n, nd, ns, b1, w2):
    npad, d = p.shape[1], p.shape[2]

    def body(p_ref, nd_ref, ns_ref, b1_ref, w2_ref, z2_ref):
        agg = p_ref[0, :n, :] + p_ref[1, :n, :]
        h = _gelu(agg * nd_ref[...][:, None] + b1_ref[...])
        y = jnp.dot(h, w2_ref[...], preferred_element_type=jnp.float32)
        z2_ref[...] = jnp.concatenate(
            [y * ns_ref[...][:, None], jnp.zeros((npad - n, d), jnp.float32)],
            axis=0)

    return pl.pallas_call(
        body,
        out_shape=jax.ShapeDtypeStruct((npad, d), jnp.float32),
    )(p, nd, ns, b1, w2)


def _tc3(p, n, nd, b2, wd, bd, k_static):
    d = p.shape[2]

    def body(p_ref, nd_ref, b2_ref, wd_ref, bd_ref, out_ref):
        agg = p_ref[0, :n, :] + p_ref[1, :n, :]
        h = _gelu(agg * nd_ref[...][:, None] + b2_ref[...])
        xr = h.reshape(n // k_static, k_static * d)
        out_ref[...] = jnp.dot(xr, wd_ref[...],
                               preferred_element_type=jnp.float32) + bd_ref[...]

    return pl.pallas_call(
        body,
        out_shape=jax.ShapeDtypeStruct((n // k_static, 1), jnp.float32),
    )(p, nd, b2, wd, bd)


def kernel(x, edge_index, k, W1, b1, W2, b2, Wd, bd):
    n, d = x.shape
    e = edge_index.shape[1]
    npad = _pad_nodes(n)
    k_static = Wd.shape[0] // d

    e_pad = NTILES * NOB * G * CS
    dummy = jnp.full((e_pad - e,), npad - 1, jnp.int32)
    src4 = jnp.concatenate([edge_index[0], dummy]).reshape(NTILES, NOB, G, CS)
    dst4 = jnp.concatenate([edge_index[1], dummy]).reshape(NTILES, NOB, G, CS)
    ones_h = jnp.ones((CS,), jnp.float32)
    zeros2_h = jnp.zeros((128, d), jnp.float32)

    do0, do1, di0, di1 = _sc_degrees(src4, dst4, ones_h, _pad_nodes_deg(n))
    z1, ns, nd = _tc1(do0, do1, di0, di1, x, W1, npad)
    p1 = _sc_spmm(z1, src4, dst4, zeros2_h, npad)
    z2 = _tc2(p1, n, nd, ns, b1.reshape(1, d), W2)
    p2 = _sc_spmm(z2, src4, dst4, zeros2_h, npad)
    out = _tc3(p2, n, nd, b2.reshape(1, d), Wd, bd.reshape(1, 1), k_static)
    return jnp.where(k == k_static, out, jnp.full_like(out, jnp.nan))


# speedup vs baseline: 3.1870x; 3.1870x over previous
import functools

import jax
import jax.numpy as jnp
from jax import lax
from jax.experimental import pallas as pl
from jax.experimental.pallas import tpu as pltpu
from jax.experimental.pallas import tpu_sc as plsc

NSC = 2
NSUB = 16
NTILES = NSC * NSUB
CS = 128
G = 4
NOB = 20


def _pad_nodes(n):
    return ((n + 128) // 128) * 128


def _pad_nodes_deg(n):
    return ((n + 255) // 256) * 256


def _mesh():
    return plsc.VectorSubcoreMesh(core_axis_name="c", subcore_axis_name="s")


def _sc_degrees(src4, dst4, ones_h, npad):
    rows_pt = npad // NSUB

    @functools.partial(
        pl.kernel,
        out_type=(jax.ShapeDtypeStruct((npad,), jnp.float32),
                  jax.ShapeDtypeStruct((npad,), jnp.float32),
                  jax.ShapeDtypeStruct((npad,), jnp.float32),
                  jax.ShapeDtypeStruct((npad,), jnp.float32)),
        mesh=_mesh(),
        scratch_types=[
            pltpu.VMEM((G, CS), jnp.int32),
            pltpu.VMEM((G, CS), jnp.int32),
            pltpu.VMEM((G, CS), jnp.int32),
            pltpu.VMEM((G, CS), jnp.int32),
            pltpu.VMEM((CS,), jnp.float32),
            pltpu.VMEM((((npad // NSUB + 15) // 16) * 16,), jnp.float32),
            pltpu.VMEM_SHARED((npad,), jnp.float32),
            pltpu.VMEM_SHARED((npad,), jnp.float32),
            pltpu.SemaphoreType.DMA,
            pltpu.SemaphoreType.DMA,
            pltpu.SemaphoreType.DMA,
        ],
    )
    def deg_kernel(src_hbm, dst_hbm, ones_hbm,
                   dout0_hbm, dout1_hbm, din0_hbm, din1_hbm,
                   sidx0, didx0, sidx1, didx1, ones, zbuf, acc_o, acc_i,
                   isem0, isem1, ssem):
        c = lax.axis_index("c")
        s = lax.axis_index("s")
        t = s * NSC + c
        idxp = ((sidx0, didx0, isem0), (sidx1, didx1, isem1))
        pltpu.async_copy(src_hbm.at[t, 0], sidx0, isem0)
        pltpu.async_copy(dst_hbm.at[t, 0], didx0, isem0)
        pltpu.async_copy(src_hbm.at[t, 1], sidx1, isem1)
        pltpu.async_copy(dst_hbm.at[t, 1], didx1, isem1)
        pltpu.sync_copy(ones_hbm, ones)

        def init_z(i, _):
            zbuf[pl.ds(i * 16, 16)] = jnp.zeros((16,), jnp.float32)
            return 0
        lax.fori_loop(0, zbuf.shape[0] // 16, init_z, 0)
        pltpu.sync_copy(zbuf.at[pl.ds(0, rows_pt)],
                        acc_o.at[pl.ds(s * rows_pt, rows_pt)])
        pltpu.sync_copy(zbuf.at[pl.ds(0, rows_pt)],
                        acc_i.at[pl.ds(s * rows_pt, rows_pt)])
        plsc.subcore_barrier()

        def process_block(ob, p):
            sidx, didx, isem = idxp[p]
            pltpu.make_async_copy(src_hbm.at[t, 0], sidx, isem).wait()
            pltpu.make_async_copy(dst_hbm.at[t, 0], didx, isem).wait()
            for g in range(G):
                pltpu.async_copy(ones, acc_o.at[sidx.at[g]], ssem, add=True)
                pltpu.async_copy(ones, acc_i.at[didx.at[g]], ssem, add=True)
            for g in range(G):
                pltpu.make_async_copy(ones, acc_o.at[sidx.at[0]], ssem).wait()
                pltpu.make_async_copy(ones, acc_i.at[didx.at[0]], ssem).wait()
            @pl.when(ob + 2 < NOB)
            def _():
                pltpu.async_copy(src_hbm.at[t, ob + 2], sidx, isem)
                pltpu.async_copy(dst_hbm.at[t, ob + 2], didx, isem)

        def body(i, _):
            process_block(2 * i, 0)
            process_block(2 * i + 1, 1)
            return 0
        lax.fori_loop(0, NOB // 2, body, 0)
        plsc.subcore_barrier()

        @pl.when(c == 0)
        def _():
            pltpu.sync_copy(acc_o.at[pl.ds(s * rows_pt, rows_pt)],
                            dout0_hbm.at[pl.ds(s * rows_pt, rows_pt)])
            pltpu.sync_copy(acc_i.at[pl.ds(s * rows_pt, rows_pt)],
                            din0_hbm.at[pl.ds(s * rows_pt, rows_pt)])

        @pl.when(c == 1)
        def _():
            pltpu.sync_copy(acc_o.at[pl.ds(s * rows_pt, rows_pt)],
                            dout1_hbm.at[pl.ds(s * rows_pt, rows_pt)])
            pltpu.sync_copy(acc_i.at[pl.ds(s * rows_pt, rows_pt)],
                            din1_hbm.at[pl.ds(s * rows_pt, rows_pt)])

    return deg_kernel(src4, dst4, ones_h)


def _sc_spmm(z, src4, dst4, zeros2_h, npad):
    d = z.shape[1]
    rows_pt = npad // NSUB

    @functools.partial(
        pl.kernel,
        out_type=jax.ShapeDtypeStruct((NSC, npad, d), jnp.float32),
        mesh=_mesh(),
        scratch_types=[
            pltpu.VMEM((G, CS), jnp.int32),
            pltpu.VMEM((G, CS), jnp.int32),
            pltpu.VMEM((G, CS), jnp.int32),
            pltpu.VMEM((G, CS), jnp.int32),
            pltpu.VMEM((G, CS), jnp.int32),
            pltpu.VMEM((G, CS), jnp.int32),
            pltpu.VMEM((G, CS), jnp.int32),
            pltpu.VMEM((G, CS), jnp.int32),
            pltpu.VMEM((CS, d), jnp.float32),
            pltpu.VMEM((CS, d), jnp.float32),
            pltpu.VMEM_SHARED((npad, d), jnp.float32),
            pltpu.SemaphoreType.DMA,
            pltpu.SemaphoreType.DMA,
            pltpu.SemaphoreType.DMA,
            pltpu.SemaphoreType.DMA,
            pltpu.SemaphoreType.DMA,
            pltpu.SemaphoreType.DMA,
            pltpu.SemaphoreType.DMA,
            pltpu.SemaphoreType.DMA,
        ],
    )
    def spmm_kernel(z_hbm, src_hbm, dst_hbm, zeros_hbm, out_hbm,
                    sidx0, didx0, sidx1, didx1, sidx2, didx2, sidx3, didx3,
                    rb0, rb1, acc,
                    sem0, sem1, isem0, isem1, isem2, isem3, ssem0, ssem1):
        c = lax.axis_index("c")
        s = lax.axis_index("s")
        t = s * NSC + c
        rbs = (rb0, rb1)
        sems = (sem0, sem1)
        ssems = (ssem0, ssem1)
        idxp = ((sidx0, didx0, isem0), (sidx1, didx1, isem1),
                (sidx2, didx2, isem2), (sidx3, didx3, isem3))

        def wait_scatter(b):
            pltpu.make_async_copy(rbs[b], acc.at[didx0.at[0]],
                                  ssems[b]).wait()

        for b in range(3):
            pltpu.async_copy(src_hbm.at[t, b], idxp[b][0], idxp[b][2])
            pltpu.async_copy(dst_hbm.at[t, b], idxp[b][1], idxp[b][2])

        def zc(i, _):
            pltpu.sync_copy(zeros_hbm,
                            acc.at[pl.ds(s * rows_pt + i * 128, 128)])
            return 0
        lax.fori_loop(0, rows_pt // 128, zc, 0)
        rem = rows_pt % 128
        if rem:
            pltpu.sync_copy(
                zeros_hbm.at[pl.ds(0, rem)],
                acc.at[pl.ds(s * rows_pt + (rows_pt - rem), rem)])
        plsc.subcore_barrier()

        def wait_idx(sidx, didx, isem):
            pltpu.make_async_copy(src_hbm.at[t, 0], sidx, isem).wait()
            pltpu.make_async_copy(dst_hbm.at[t, 0], didx, isem).wait()

        def process_block(ob, p):
            sidx, didx, isem = idxp[p]
            wait_idx(sidx, didx, isem)

            @pl.when(ob > 0)
            def _():
                wait_scatter(0)
            cps = [None, None]
            cps[0] = pltpu.async_copy(z_hbm.at[sidx.at[0]], rb0, sem0)
            for g in range(G):
                b = g % 2
                if g + 1 < G:
                    nb = (g + 1) % 2
                    if g == 0:
                        @pl.when(ob > 0)
                        def _():
                            wait_scatter(1)
                    else:
                        wait_scatter(nb)
                    cps[nb] = pltpu.async_copy(
                        z_hbm.at[sidx.at[g + 1]], rbs[nb], sems[nb])
                cps[b].wait()
                pltpu.async_copy(rbs[b], acc.at[didx.at[g]], ssems[b],
                                 add=True)
            @pl.when(ob + 3 < NOB)
            def _():
                rsidx, rdidx, risem = idxp[(p + 3) % 4]
                pltpu.async_copy(src_hbm.at[t, ob + 3], rsidx, risem)
                pltpu.async_copy(dst_hbm.at[t, ob + 3], rdidx, risem)

        def body(i, _):
            for j in range(4):
                process_block(4 * i + j, j)
            return 0
        lax.fori_loop(0, NOB // 4, body, 0)
        wait_scatter(0)
        wait_scatter(1)
        plsc.subcore_barrier()
        pltpu.sync_copy(acc.at[pl.ds(s * rows_pt, rows_pt)],
                        out_hbm.at[c, pl.ds(s * rows_pt, rows_pt)])

    return spmm_kernel(z, src4, dst4, zeros2_h)


def _gelu(x):
    return 0.5 * x * (1.0 + lax.erf(x * (2.0 ** -0.5)))


def _tc1(do0, do1, di0, di1, x, w1, npad):
    n, d = x.shape

    def body(do0_ref, do1_ref, di0_ref, di1_ref, x_ref, w1_ref,
             z1_ref, ns_ref, nd_ref):
        od = do0_ref[...] + do1_ref[...]
        idg = di0_ref[...] + di1_ref[...]
        ns = jnp.where(od > 0, lax.rsqrt(jnp.maximum(od, 1.0)), 0.0)[:n]
        nd = jnp.where(idg > 0, lax.rsqrt(jnp.maximum(idg, 1.0)), 0.0)[:n]
        y = jnp.dot(x_ref[...], w1_ref[...], preferred_element_type=jnp.float32)
        z1_ref[...] = jnp.concatenate(
            [y * ns[:, None], jnp.zeros((npad - n, d), jnp.float32)], axis=0)
        ns_ref[...] = ns
        nd_ref[...] = nd

    return pl.pallas_call(
        body,
        out_shape=(jax.ShapeDtypeStruct((npad, d), jnp.float32),
                   jax.ShapeDtypeStruct((n,), jnp.float32),
                   jax.ShapeDtypeStruct((n,), jnp.float32)),
    )(do0, do1, di0, di1, x, w1)


def _tc2(p, n, nd, ns, b1, w2):
    npad, d = p.shape[1], p.shape[2]

    def body(p_ref, nd_ref, ns_ref, b1_ref, w2_ref, z2_ref):
        agg = p_ref[0, :n, :] + p_ref[1, :n, :]
        h = _gelu(agg * nd_ref[...][:, None] + b1_ref[...])
        y = jnp.dot(h, w2_ref[...], preferred_element_type=jnp.float32)
        z2_ref[...] = jnp.concatenate(
            [y * ns_ref[...][:, None], jnp.zeros((npad - n, d), jnp.float32)],
            axis=0)

    return pl.pallas_call(
        body,
        out_shape=jax.ShapeDtypeStruct((npad, d), jnp.float32),
    )(p, nd, ns, b1, w2)


def _tc3(p, n, nd, b2, wd, bd, k_static):
    d = p.shape[2]

    def body(p_ref, nd_ref, b2_ref, wd_ref, bd_ref, out_ref):
        agg = p_ref[0, :n, :] + p_ref[1, :n, :]
        h = _gelu(agg * nd_ref[...][:, None] + b2_ref[...])
        xr = h.reshape(n // k_static, k_static * d)
        out_ref[...] = jnp.dot(xr, wd_ref[...],
                               preferred_element_type=jnp.float32) + bd_ref[...]

    return pl.pallas_call(
        body,
        out_shape=jax.ShapeDtypeStruct((n // k_static, 1), jnp.float32),
    )(p, nd, b2, wd, bd)


def kernel(x, edge_index, k, W1, b1, W2, b2, Wd, bd):
    n, d = x.shape
    e = edge_index.shape[1]
    npad = _pad_nodes(n)
    k_static = Wd.shape[0] // d

    e_pad = NTILES * NOB * G * CS
    dummy = n + (jnp.arange(e_pad - e, dtype=jnp.int32) % (npad - n))
    src4 = jnp.concatenate([edge_index[0], dummy]).reshape(NTILES, NOB, G, CS)
    dst4 = jnp.concatenate([edge_index[1], dummy]).reshape(NTILES, NOB, G, CS)
    ones_h = jnp.ones((CS,), jnp.float32)
    zeros2_h = jnp.zeros((128, d), jnp.float32)

    do0, do1, di0, di1 = _sc_degrees(src4, dst4, ones_h, _pad_nodes_deg(n))
    z1, ns, nd = _tc1(do0, do1, di0, di1, x, W1, npad)
    p1 = _sc_spmm(z1, src4, dst4, zeros2_h, npad)
    z2 = _tc2(p1, n, nd, ns, b1.reshape(1, d), W2)
    p2 = _sc_spmm(z2, src4, dst4, zeros2_h, npad)
    out = _tc3(p2, n, nd, b2.reshape(1, d), Wd, bd.reshape(1, 1), k_static)
    return jnp.where(k == k_static, out, jnp.full_like(out, jnp.nan))
